# baseline (speedup 1.0000x reference)
import functools

import jax
import jax.numpy as jnp
from jax import lax
from jax.experimental import pallas as pl
from jax.experimental.pallas import tpu as pltpu
from jax.experimental.pallas import tpu_sc as plsc

N = 10000
E = 320000
IN_DIM = 128
HID = 256
REPR = 128
BK = 1000
G = N // BK

NC = 2
NS = 16
EB = 80
NITER = E // (NS * EB)
CH = 50
NPAD = 10240
ROWS_PER_SUB = NPAD // NS

_DOT = functools.partial(jnp.dot, preferred_element_type=jnp.float32)


def _row_spec(d):
    return pl.BlockSpec((BK, d), lambda i: (i, 0))


def _full_spec(shape):
    nd = len(shape)
    return pl.BlockSpec(shape, lambda i: (0,) * nd)


def _mlp0_body(x_ref, agg_ref, w1_ref, b1_ref, w2_ref, b2_ref,
               hl_ref, hr_ref):
    m = x_ref[...] + agg_ref[...]
    a = jnp.maximum(_DOT(m, w1_ref[...]) + b1_ref[...], 0.0)
    h = jnp.maximum(_DOT(a, w2_ref[...]) + b2_ref[...], 0.0)
    hl_ref[...] = h[:, :HID // 2]
    hr_ref[...] = h[:, HID // 2:]


def _mlp0(x, agg, w1, b1, w2, b2):
    return pl.pallas_call(
        _mlp0_body,
        grid=(G,),
        in_specs=[_row_spec(IN_DIM), _row_spec(IN_DIM),
                  _full_spec((IN_DIM, HID)), _full_spec((1, HID)),
                  _full_spec((HID, HID)), _full_spec((1, HID))],
        out_specs=[_row_spec(HID // 2), _row_spec(HID // 2)],
        out_shape=[jax.ShapeDtypeStruct((N, HID // 2), jnp.float32),
                   jax.ShapeDtypeStruct((N, HID // 2), jnp.float32)],
    )(x, agg, w1, b1.reshape(1, -1), w2, b2.reshape(1, -1))


def _mlp1_body(hl_ref, hr_ref, al_ref, ar_ref, w1_ref, b1_ref, w2_ref,
               b2_ref, out_ref):
    m = jnp.concatenate([hl_ref[...] + al_ref[...],
                         hr_ref[...] + ar_ref[...]], axis=1)
    a = jnp.maximum(_DOT(m, w1_ref[...]) + b1_ref[...], 0.0)
    out_ref[...] = _DOT(a, w2_ref[...]) + b2_ref[...]


def _mlp1(hl, hr, al, ar, w1, b1, w2, b2):
    return pl.pallas_call(
        _mlp1_body,
        grid=(G,),
        in_specs=[_row_spec(HID // 2)] * 4 +
                 [_full_spec((HID, HID)), _full_spec((1, HID)),
                  _full_spec((HID, REPR)), _full_spec((1, REPR))],
        out_specs=_row_spec(REPR),
        out_shape=jax.ShapeDtypeStruct((N, REPR), jnp.float32),
    )(hl, hr, al, ar, w1, b1.reshape(1, -1), w2, b2.reshape(1, -1))


def _proj_a_body(h_ref, w1_ref, b1_ref, t_ref, s1_ref, s2_ref):
    t = _DOT(h_ref[...], w1_ref[...]) + b1_ref[...]
    t_ref[...] = t
    s1_ref[...] = jnp.sum(t, axis=0, keepdims=True)[None]
    s2_ref[...] = jnp.sum(t * t, axis=0, keepdims=True)[None]


def _proj_a(h, w1, b1):
    return pl.pallas_call(
        _proj_a_body,
        grid=(G,),
        in_specs=[_row_spec(REPR), _full_spec((REPR, REPR)),
                  _full_spec((1, REPR))],
        out_specs=[_row_spec(REPR),
                   pl.BlockSpec((1, 1, REPR), lambda i: (i, 0, 0)),
                   pl.BlockSpec((1, 1, REPR), lambda i: (i, 0, 0))],
        out_shape=[jax.ShapeDtypeStruct((N, REPR), jnp.float32),
                   jax.ShapeDtypeStruct((G, 1, REPR), jnp.float32),
                   jax.ShapeDtypeStruct((G, 1, REPR), jnp.float32)],
    )(h, w1, b1.reshape(1, -1))


def _proj_b_body(t_ref, s1_ref, s2_ref, gamma_ref, beta_ref, w2_ref,
                 b2_ref, z_ref):
    mean = jnp.sum(s1_ref[...], axis=0) / N
    ex2 = jnp.sum(s2_ref[...], axis=0) / N
    var = ex2 - mean * mean
    norm = gamma_ref[...] * (t_ref[...] - mean) * jax.lax.rsqrt(var + 1e-5) \
        + beta_ref[...]
    z_ref[...] = _DOT(jnp.maximum(norm, 0.0), w2_ref[...]) + b2_ref[...]


def _proj_b(t, s1, s2, gamma, beta, w2, b2):
    return pl.pallas_call(
        _proj_b_body,
        grid=(G,),
        in_specs=[_row_spec(REPR), _full_spec((G, 1, REPR)),
                  _full_spec((G, 1, REPR)), _full_spec((1, REPR)),
                  _full_spec((1, REPR)), _full_spec((REPR, REPR)),
                  _full_spec((1, REPR))],
        out_specs=_row_spec(REPR),
        out_shape=jax.ShapeDtypeStruct((N, REPR), jnp.float32),
    )(t, s1, s2, gamma.reshape(1, -1), beta.reshape(1, -1), w2,
      b2.reshape(1, -1))


def _proj_head(p, h):
    t, s1, s2 = _proj_a(h, p['W1'], p['b1'])
    return _proj_b(t, s1, s2, p['gamma'], p['beta'], p['W2'], p['b2'])


def _make_sc_agg(nrounds):
    n_tab = NC * nrounds
    mesh = plsc.VectorSubcoreMesh(core_axis_name="c", subcore_axis_name="s")

    def body(*refs):
        tabs = refs[:n_tab]
        src_hbm, dst_hbm, zeros_hbm = refs[n_tab:n_tab + 3]
        outs = refs[n_tab + 3:2 * n_tab + 3]
        (acc, src_v, dst_v, rows_v, rows_b, zbuf, sem, sem_b, sem_sa,
         sem_sb) = refs[2 * n_tab + 3:]
        c = lax.axis_index("c")
        s = lax.axis_index("s")
        row0 = s * ROWS_PER_SUB
        pltpu.sync_copy(zeros_hbm, zbuf)
        nchunk = ROWS_PER_SUB // EB
        for r in range(nrounds):
            for k in range(nchunk):
                pltpu.sync_copy(zbuf, acc.at[pl.ds(row0 + k * EB, EB)])
            plsc.subcore_barrier()
            for ci in range(NC):
                tab = tabs[r * NC + ci]

                @pl.when(c == ci)
                def _():
                    def chunk(g, carry):
                        pltpu.sync_copy(src_hbm.at[s, pl.ds(g * CH, CH)],
                                        src_v)
                        pltpu.sync_copy(dst_hbm.at[s, pl.ds(g * CH, CH)],
                                        dst_v)
                        pltpu.async_copy(tab.at[src_v.at[0]], rows_v, sem)

                        def it(i, carry2):
                            j0 = 2 * i
                            pltpu.make_async_copy(
                                tab.at[src_v.at[j0]], rows_v, sem).wait()

                            @pl.when(i > 0)
                            def _():
                                pltpu.make_async_copy(
                                    rows_b, acc.at[dst_v.at[j0 - 1]],
                                    sem_sb).wait()
                            pltpu.async_copy(rows_v,
                                             acc.at[dst_v.at[j0]],
                                             sem_sa, add=True)
                            pltpu.async_copy(
                                tab.at[src_v.at[j0 + 1]], rows_b, sem_b)
                            pltpu.make_async_copy(
                                tab.at[src_v.at[j0 + 1]], rows_b,
                                sem_b).wait()
                            pltpu.async_copy(rows_b,
                                             acc.at[dst_v.at[j0 + 1]],
                                             sem_sb, add=True)
                            pltpu.make_async_copy(
                                rows_v, acc.at[dst_v.at[j0]],
                                sem_sa).wait()

                            @pl.when(i < CH // 2 - 1)
                            def _():
                                pltpu.async_copy(
                                    tab.at[src_v.at[j0 + 2]], rows_v, sem)
                            return carry2
                        lax.fori_loop(0, CH // 2, it, 0)
                        pltpu.make_async_copy(
                            rows_b, acc.at[dst_v.at[CH - 1]],
                            sem_sb).wait()
                        return carry
                    lax.fori_loop(0, NITER // CH, chunk, 0)
            plsc.subcore_barrier()
            for ci in range(NC):
                out = outs[r * NC + ci]

                @pl.when(c == ci)
                def _():
                    for k in range(nchunk):
                        pltpu.sync_copy(acc.at[pl.ds(row0 + k * EB, EB)],
                                        rows_v)
                        pltpu.sync_copy(rows_v,
                                        out.at[pl.ds(row0 + k * EB, EB)])
            if r + 1 < nrounds:
                plsc.subcore_barrier()

    return pl.kernel(
        body,
        out_type=[jax.ShapeDtypeStruct((NPAD, 128), jnp.float32)] * n_tab,
        mesh=mesh,
        compiler_params=pltpu.CompilerParams(use_tc_tiling_on_sc=False),
        scratch_types=[
            pltpu.VMEM_SHARED((NPAD, 128), jnp.float32),
            pltpu.VMEM((CH, EB), jnp.int32),
            pltpu.VMEM((CH, EB), jnp.int32),
            pltpu.VMEM((EB, 128), jnp.float32),
            pltpu.VMEM((EB, 128), jnp.float32),
            pltpu.VMEM((EB, 128), jnp.float32),
            pltpu.SemaphoreType.DMA,
            pltpu.SemaphoreType.DMA,
            pltpu.SemaphoreType.DMA,
            pltpu.SemaphoreType.DMA,
        ],
    )


_sc_agg2 = _make_sc_agg(1)
_sc_agg4 = _make_sc_agg(2)


def kernel(x_phys, x_sem, edge_index, params):
    src3 = edge_index[0].reshape(NS, NITER, EB)
    dst3 = edge_index[1].reshape(NS, NITER, EB)
    zeros = jnp.zeros((EB, 128), jnp.float32)
    pp = params['phys_enc']
    ps = params['sem_enc']

    agg0_p, agg0_s = _sc_agg2(x_phys, x_sem, src3, dst3, zeros)
    h1l_p, h1r_p = _mlp0(x_phys, agg0_p[:N], pp[0]['W1'], pp[0]['b1'],
                         pp[0]['W2'], pp[0]['b2'])
    h1l_s, h1r_s = _mlp0(x_sem, agg0_s[:N], ps[0]['W1'], ps[0]['b1'],
                         ps[0]['W2'], ps[0]['b2'])
    a1l_p, a1l_s, a1r_p, a1r_s = (
        a[:N] for a in _sc_agg4(h1l_p, h1l_s, h1r_p, h1r_s,
                                src3, dst3, zeros))
    h2_p = _mlp1(h1l_p, h1r_p, a1l_p, a1r_p, pp[1]['W1'], pp[1]['b1'],
                 pp[1]['W2'], pp[1]['b2'])
    h2_s = _mlp1(h1l_s, h1r_s, a1l_s, a1r_s, ps[1]['W1'], ps[1]['b1'],
                 ps[1]['W2'], ps[1]['b2'])
    z_p = _proj_head(params['phys_proj'], h2_p)
    z_s = _proj_head(params['sem_proj'], h2_s)
    return (h2_p, h2_s, z_p, z_s)


# speedup vs baseline: 1.0859x; 1.0859x over previous
import functools

import jax
import jax.numpy as jnp
from jax import lax
from jax.experimental import pallas as pl
from jax.experimental.pallas import tpu as pltpu
from jax.experimental.pallas import tpu_sc as plsc

N = 10000
E = 320000
IN_DIM = 128
HID = 256
REPR = 128
BK = 1000
G = N // BK

NC = 2
NS = 16
EB = 100
NITER = E // (NS * EB)
CH = 50
ZB = 80
NPAD = 10240
ROWS_PER_SUB = NPAD // NS

_DOT = functools.partial(jnp.dot, preferred_element_type=jnp.float32)


def _row_spec(d):
    return pl.BlockSpec((BK, d), lambda i: (i, 0))


def _full_spec(shape):
    nd = len(shape)
    return pl.BlockSpec(shape, lambda i: (0,) * nd)


def _mlp0_body(x_ref, agg_ref, w1_ref, b1_ref, w2_ref, b2_ref,
               hl_ref, hr_ref):
    m = x_ref[...] + agg_ref[...]
    a = jnp.maximum(_DOT(m, w1_ref[...]) + b1_ref[...], 0.0)
    h = jnp.maximum(_DOT(a, w2_ref[...]) + b2_ref[...], 0.0)
    hl_ref[...] = h[:, :HID // 2]
    hr_ref[...] = h[:, HID // 2:]


def _mlp0(x, agg, w1, b1, w2, b2):
    return pl.pallas_call(
        _mlp0_body,
        grid=(G,),
        in_specs=[_row_spec(IN_DIM), _row_spec(IN_DIM),
                  _full_spec((IN_DIM, HID)), _full_spec((1, HID)),
                  _full_spec((HID, HID)), _full_spec((1, HID))],
        out_specs=[_row_spec(HID // 2), _row_spec(HID // 2)],
        out_shape=[jax.ShapeDtypeStruct((N, HID // 2), jnp.float32),
                   jax.ShapeDtypeStruct((N, HID // 2), jnp.float32)],
    )(x, agg, w1, b1.reshape(1, -1), w2, b2.reshape(1, -1))


def _mlp1_body(hl_ref, hr_ref, al_ref, ar_ref, w1_ref, b1_ref, w2_ref,
               b2_ref, out_ref):
    m = jnp.concatenate([hl_ref[...] + al_ref[...],
                         hr_ref[...] + ar_ref[...]], axis=1)
    a = jnp.maximum(_DOT(m, w1_ref[...]) + b1_ref[...], 0.0)
    out_ref[...] = _DOT(a, w2_ref[...]) + b2_ref[...]


def _mlp1(hl, hr, al, ar, w1, b1, w2, b2):
    return pl.pallas_call(
        _mlp1_body,
        grid=(G,),
        in_specs=[_row_spec(HID // 2)] * 4 +
                 [_full_spec((HID, HID)), _full_spec((1, HID)),
                  _full_spec((HID, REPR)), _full_spec((1, REPR))],
        out_specs=_row_spec(REPR),
        out_shape=jax.ShapeDtypeStruct((N, REPR), jnp.float32),
    )(hl, hr, al, ar, w1, b1.reshape(1, -1), w2, b2.reshape(1, -1))


def _proj_a_body(h_ref, w1_ref, b1_ref, t_ref, s1_ref, s2_ref):
    t = _DOT(h_ref[...], w1_ref[...]) + b1_ref[...]
    t_ref[...] = t
    s1_ref[...] = jnp.sum(t, axis=0, keepdims=True)[None]
    s2_ref[...] = jnp.sum(t * t, axis=0, keepdims=True)[None]


def _proj_a(h, w1, b1):
    return pl.pallas_call(
        _proj_a_body,
        grid=(G,),
        in_specs=[_row_spec(REPR), _full_spec((REPR, REPR)),
                  _full_spec((1, REPR))],
        out_specs=[_row_spec(REPR),
                   pl.BlockSpec((1, 1, REPR), lambda i: (i, 0, 0)),
                   pl.BlockSpec((1, 1, REPR), lambda i: (i, 0, 0))],
        out_shape=[jax.ShapeDtypeStruct((N, REPR), jnp.float32),
                   jax.ShapeDtypeStruct((G, 1, REPR), jnp.float32),
                   jax.ShapeDtypeStruct((G, 1, REPR), jnp.float32)],
    )(h, w1, b1.reshape(1, -1))


def _proj_b_body(t_ref, s1_ref, s2_ref, gamma_ref, beta_ref, w2_ref,
                 b2_ref, z_ref):
    mean = jnp.sum(s1_ref[...], axis=0) / N
    ex2 = jnp.sum(s2_ref[...], axis=0) / N
    var = ex2 - mean * mean
    norm = gamma_ref[...] * (t_ref[...] - mean) * jax.lax.rsqrt(var + 1e-5) \
        + beta_ref[...]
    z_ref[...] = _DOT(jnp.maximum(norm, 0.0), w2_ref[...]) + b2_ref[...]


def _proj_b(t, s1, s2, gamma, beta, w2, b2):
    return pl.pallas_call(
        _proj_b_body,
        grid=(G,),
        in_specs=[_row_spec(REPR), _full_spec((G, 1, REPR)),
                  _full_spec((G, 1, REPR)), _full_spec((1, REPR)),
                  _full_spec((1, REPR)), _full_spec((REPR, REPR)),
                  _full_spec((1, REPR))],
        out_specs=_row_spec(REPR),
        out_shape=jax.ShapeDtypeStruct((N, REPR), jnp.float32),
    )(t, s1, s2, gamma.reshape(1, -1), beta.reshape(1, -1), w2,
      b2.reshape(1, -1))


def _proj_head(p, h):
    t, s1, s2 = _proj_a(h, p['W1'], p['b1'])
    return _proj_b(t, s1, s2, p['gamma'], p['beta'], p['W2'], p['b2'])


def _make_sc_agg(nrounds):
    n_tab = NC * nrounds
    mesh = plsc.VectorSubcoreMesh(core_axis_name="c", subcore_axis_name="s")

    def body(*refs):
        tabs = refs[:n_tab]
        src_hbm, dst_hbm, zeros_hbm = refs[n_tab:n_tab + 3]
        outs = refs[n_tab + 3:2 * n_tab + 3]
        (acc, src_v, dst_v, rows_v, rows_b, zbuf, sem, sem_b, sem_sa,
         sem_sb) = refs[2 * n_tab + 3:]
        c = lax.axis_index("c")
        s = lax.axis_index("s")
        row0 = s * ROWS_PER_SUB
        pltpu.sync_copy(zeros_hbm, zbuf)
        nchunk = ROWS_PER_SUB // ZB
        for r in range(nrounds):
            for k in range(nchunk):
                pltpu.sync_copy(zbuf, acc.at[pl.ds(row0 + k * ZB, ZB)])
            plsc.subcore_barrier()
            for ci in range(NC):
                tab = tabs[r * NC + ci]

                @pl.when(c == ci)
                def _():
                    def chunk(g, carry):
                        pltpu.sync_copy(src_hbm.at[s, pl.ds(g * CH, CH)],
                                        src_v)
                        pltpu.sync_copy(dst_hbm.at[s, pl.ds(g * CH, CH)],
                                        dst_v)
                        pltpu.async_copy(tab.at[src_v.at[0]], rows_v, sem)

                        def it(i, carry2):
                            j0 = 2 * i
                            pltpu.make_async_copy(
                                tab.at[src_v.at[j0]], rows_v, sem).wait()

                            @pl.when(i > 0)
                            def _():
                                pltpu.make_async_copy(
                                    rows_b, acc.at[dst_v.at[j0 - 1]],
                                    sem_sb).wait()
                            pltpu.async_copy(rows_v,
                                             acc.at[dst_v.at[j0]],
                                             sem_sa, add=True)
                            pltpu.async_copy(
                                tab.at[src_v.at[j0 + 1]], rows_b, sem_b)
                            pltpu.make_async_copy(
                                tab.at[src_v.at[j0 + 1]], rows_b,
                                sem_b).wait()
                            pltpu.async_copy(rows_b,
                                             acc.at[dst_v.at[j0 + 1]],
                                             sem_sb, add=True)
                            pltpu.make_async_copy(
                                rows_v, acc.at[dst_v.at[j0]],
                                sem_sa).wait()

                            @pl.when(i < CH // 2 - 1)
                            def _():
                                pltpu.async_copy(
                                    tab.at[src_v.at[j0 + 2]], rows_v, sem)
                            return carry2
                        lax.fori_loop(0, CH // 2, it, 0)
                        pltpu.make_async_copy(
                            rows_b, acc.at[dst_v.at[CH - 1]],
                            sem_sb).wait()
                        return carry
                    lax.fori_loop(0, NITER // CH, chunk, 0)
            plsc.subcore_barrier()
            for ci in range(NC):
                out = outs[r * NC + ci]

                @pl.when(c == ci)
                def _():
                    for k in range(nchunk):
                        pltpu.sync_copy(acc.at[pl.ds(row0 + k * ZB, ZB)],
                                        rows_v.at[pl.ds(0, ZB)])
                        pltpu.sync_copy(rows_v.at[pl.ds(0, ZB)],
                                        out.at[pl.ds(row0 + k * ZB, ZB)])
            if r + 1 < nrounds:
                plsc.subcore_barrier()

    return pl.kernel(
        body,
        out_type=[jax.ShapeDtypeStruct((NPAD, 128), jnp.float32)] * n_tab,
        mesh=mesh,
        compiler_params=pltpu.CompilerParams(use_tc_tiling_on_sc=False),
        scratch_types=[
            pltpu.VMEM_SHARED((NPAD, 128), jnp.float32),
            pltpu.VMEM((CH, EB), jnp.int32),
            pltpu.VMEM((CH, EB), jnp.int32),
            pltpu.VMEM((EB, 128), jnp.float32),
            pltpu.VMEM((EB, 128), jnp.float32),
            pltpu.VMEM((ZB, 128), jnp.float32),
            pltpu.SemaphoreType.DMA,
            pltpu.SemaphoreType.DMA,
            pltpu.SemaphoreType.DMA,
            pltpu.SemaphoreType.DMA,
        ],
    )


_sc_agg2 = _make_sc_agg(1)
_sc_agg4 = _make_sc_agg(2)


def kernel(x_phys, x_sem, edge_index, params):
    src3 = edge_index[0].reshape(NS, NITER, EB)
    dst3 = edge_index[1].reshape(NS, NITER, EB)
    zeros = jnp.zeros((ZB, 128), jnp.float32)
    pp = params['phys_enc']
    ps = params['sem_enc']

    agg0_p, agg0_s = _sc_agg2(x_phys, x_sem, src3, dst3, zeros)
    h1l_p, h1r_p = _mlp0(x_phys, agg0_p[:N], pp[0]['W1'], pp[0]['b1'],
                         pp[0]['W2'], pp[0]['b2'])
    h1l_s, h1r_s = _mlp0(x_sem, agg0_s[:N], ps[0]['W1'], ps[0]['b1'],
                         ps[0]['W2'], ps[0]['b2'])
    a1l_p, a1l_s, a1r_p, a1r_s = (
        a[:N] for a in _sc_agg4(h1l_p, h1l_s, h1r_p, h1r_s,
                                src3, dst3, zeros))
    h2_p = _mlp1(h1l_p, h1r_p, a1l_p, a1r_p, pp[1]['W1'], pp[1]['b1'],
                 pp[1]['W2'], pp[1]['b2'])
    h2_s = _mlp1(h1l_s, h1r_s, a1l_s, a1r_s, ps[1]['W1'], ps[1]['b1'],
                 ps[1]['W2'], ps[1]['b2'])
    z_p = _proj_head(params['phys_proj'], h2_p)
    z_s = _proj_head(params['sem_proj'], h2_s)
    return (h2_p, h2_s, z_p, z_s)


# speedup vs baseline: 1.1449x; 1.0543x over previous
import functools

import jax
import jax.numpy as jnp
from jax import lax
from jax.experimental import pallas as pl
from jax.experimental.pallas import tpu as pltpu
from jax.experimental.pallas import tpu_sc as plsc

N = 10000
E = 320000
IN_DIM = 128
HID = 256
REPR = 128
BK = 1000
G = N // BK

NC = 2
NS = 16
EB = 125
NITER = E // (NS * EB)
CH = 20
ZB = 80
NPAD = 10240
ROWS_PER_SUB = NPAD // NS

_DOT = functools.partial(jnp.dot, preferred_element_type=jnp.float32)


def _row_spec(d):
    return pl.BlockSpec((BK, d), lambda i: (i, 0))


def _full_spec(shape):
    nd = len(shape)
    return pl.BlockSpec(shape, lambda i: (0,) * nd)


def _mlp0_body(x_ref, agg_ref, w1_ref, b1_ref, w2_ref, b2_ref,
               hl_ref, hr_ref):
    m = x_ref[...] + agg_ref[...]
    a = jnp.maximum(_DOT(m, w1_ref[...]) + b1_ref[...], 0.0)
    h = jnp.maximum(_DOT(a, w2_ref[...]) + b2_ref[...], 0.0)
    hl_ref[...] = h[:, :HID // 2]
    hr_ref[...] = h[:, HID // 2:]


def _mlp0(x, agg, w1, b1, w2, b2):
    return pl.pallas_call(
        _mlp0_body,
        grid=(G,),
        in_specs=[_row_spec(IN_DIM), _row_spec(IN_DIM),
                  _full_spec((IN_DIM, HID)), _full_spec((1, HID)),
                  _full_spec((HID, HID)), _full_spec((1, HID))],
        out_specs=[_row_spec(HID // 2), _row_spec(HID // 2)],
        out_shape=[jax.ShapeDtypeStruct((N, HID // 2), jnp.float32),
                   jax.ShapeDtypeStruct((N, HID // 2), jnp.float32)],
    )(x, agg, w1, b1.reshape(1, -1), w2, b2.reshape(1, -1))


def _mlp1_body(hl_ref, hr_ref, al_ref, ar_ref, w1_ref, b1_ref, w2_ref,
               b2_ref, out_ref):
    m = jnp.concatenate([hl_ref[...] + al_ref[...],
                         hr_ref[...] + ar_ref[...]], axis=1)
    a = jnp.maximum(_DOT(m, w1_ref[...]) + b1_ref[...], 0.0)
    out_ref[...] = _DOT(a, w2_ref[...]) + b2_ref[...]


def _mlp1(hl, hr, al, ar, w1, b1, w2, b2):
    return pl.pallas_call(
        _mlp1_body,
        grid=(G,),
        in_specs=[_row_spec(HID // 2)] * 4 +
                 [_full_spec((HID, HID)), _full_spec((1, HID)),
                  _full_spec((HID, REPR)), _full_spec((1, REPR))],
        out_specs=_row_spec(REPR),
        out_shape=jax.ShapeDtypeStruct((N, REPR), jnp.float32),
    )(hl, hr, al, ar, w1, b1.reshape(1, -1), w2, b2.reshape(1, -1))


def _proj_a_body(h_ref, w1_ref, b1_ref, t_ref, s1_ref, s2_ref):
    t = _DOT(h_ref[...], w1_ref[...]) + b1_ref[...]
    t_ref[...] = t
    s1_ref[...] = jnp.sum(t, axis=0, keepdims=True)[None]
    s2_ref[...] = jnp.sum(t * t, axis=0, keepdims=True)[None]


def _proj_a(h, w1, b1):
    return pl.pallas_call(
        _proj_a_body,
        grid=(G,),
        in_specs=[_row_spec(REPR), _full_spec((REPR, REPR)),
                  _full_spec((1, REPR))],
        out_specs=[_row_spec(REPR),
                   pl.BlockSpec((1, 1, REPR), lambda i: (i, 0, 0)),
                   pl.BlockSpec((1, 1, REPR), lambda i: (i, 0, 0))],
        out_shape=[jax.ShapeDtypeStruct((N, REPR), jnp.float32),
                   jax.ShapeDtypeStruct((G, 1, REPR), jnp.float32),
                   jax.ShapeDtypeStruct((G, 1, REPR), jnp.float32)],
    )(h, w1, b1.reshape(1, -1))


def _proj_b_body(t_ref, s1_ref, s2_ref, gamma_ref, beta_ref, w2_ref,
                 b2_ref, z_ref):
    mean = jnp.sum(s1_ref[...], axis=0) / N
    ex2 = jnp.sum(s2_ref[...], axis=0) / N
    var = ex2 - mean * mean
    norm = gamma_ref[...] * (t_ref[...] - mean) * jax.lax.rsqrt(var + 1e-5) \
        + beta_ref[...]
    z_ref[...] = _DOT(jnp.maximum(norm, 0.0), w2_ref[...]) + b2_ref[...]


def _proj_b(t, s1, s2, gamma, beta, w2, b2):
    return pl.pallas_call(
        _proj_b_body,
        grid=(G,),
        in_specs=[_row_spec(REPR), _full_spec((G, 1, REPR)),
                  _full_spec((G, 1, REPR)), _full_spec((1, REPR)),
                  _full_spec((1, REPR)), _full_spec((REPR, REPR)),
                  _full_spec((1, REPR))],
        out_specs=_row_spec(REPR),
        out_shape=jax.ShapeDtypeStruct((N, REPR), jnp.float32),
    )(t, s1, s2, gamma.reshape(1, -1), beta.reshape(1, -1), w2,
      b2.reshape(1, -1))


def _proj_head(p, h):
    t, s1, s2 = _proj_a(h, p['W1'], p['b1'])
    return _proj_b(t, s1, s2, p['gamma'], p['beta'], p['W2'], p['b2'])


def _make_sc_agg(nrounds):
    n_tab = NC * nrounds
    mesh = plsc.VectorSubcoreMesh(core_axis_name="c", subcore_axis_name="s")

    def body(*refs):
        tabs = refs[:n_tab]
        src_hbm, dst_hbm, zeros_hbm = refs[n_tab:n_tab + 3]
        outs = refs[n_tab + 3:2 * n_tab + 3]
        (acc, src_v, dst_v, rows_v, rows_b, zbuf, sem, sem_b, sem_sa,
         sem_sb) = refs[2 * n_tab + 3:]
        c = lax.axis_index("c")
        s = lax.axis_index("s")
        row0 = s * ROWS_PER_SUB
        pltpu.sync_copy(zeros_hbm, zbuf)
        nchunk = ROWS_PER_SUB // ZB
        for r in range(nrounds):
            for k in range(nchunk):
                pltpu.sync_copy(zbuf, acc.at[pl.ds(row0 + k * ZB, ZB)])
            plsc.subcore_barrier()
            for ci in range(NC):
                tab = tabs[r * NC + ci]

                @pl.when(c == ci)
                def _():
                    def chunk(g, carry):
                        pltpu.sync_copy(src_hbm.at[s, pl.ds(g * CH, CH)],
                                        src_v)
                        pltpu.sync_copy(dst_hbm.at[s, pl.ds(g * CH, CH)],
                                        dst_v)
                        pltpu.async_copy(tab.at[src_v.at[0]], rows_v, sem)

                        def it(i, carry2):
                            j0 = 2 * i
                            pltpu.make_async_copy(
                                tab.at[src_v.at[j0]], rows_v, sem).wait()

                            @pl.when(i > 0)
                            def _():
                                pltpu.make_async_copy(
                                    rows_b, acc.at[dst_v.at[j0 - 1]],
                                    sem_sb).wait()
                            pltpu.async_copy(rows_v,
                                             acc.at[dst_v.at[j0]],
                                             sem_sa, add=True)
                            pltpu.async_copy(
                                tab.at[src_v.at[j0 + 1]], rows_b, sem_b)
                            pltpu.make_async_copy(
                                tab.at[src_v.at[j0 + 1]], rows_b,
                                sem_b).wait()
                            pltpu.async_copy(rows_b,
                                             acc.at[dst_v.at[j0 + 1]],
                                             sem_sb, add=True)
                            pltpu.make_async_copy(
                                rows_v, acc.at[dst_v.at[j0]],
                                sem_sa).wait()

                            @pl.when(i < CH // 2 - 1)
                            def _():
                                pltpu.async_copy(
                                    tab.at[src_v.at[j0 + 2]], rows_v, sem)
                            return carry2
                        lax.fori_loop(0, CH // 2, it, 0)
                        pltpu.make_async_copy(
                            rows_b, acc.at[dst_v.at[CH - 1]],
                            sem_sb).wait()
                        return carry
                    lax.fori_loop(0, NITER // CH, chunk, 0)
            plsc.subcore_barrier()
            for ci in range(NC):
                out = outs[r * NC + ci]

                @pl.when(c == ci)
                def _():
                    for k in range(nchunk):
                        pltpu.sync_copy(acc.at[pl.ds(row0 + k * ZB, ZB)],
                                        rows_v.at[pl.ds(0, ZB)])
                        pltpu.sync_copy(rows_v.at[pl.ds(0, ZB)],
                                        out.at[pl.ds(row0 + k * ZB, ZB)])
            if r + 1 < nrounds:
                plsc.subcore_barrier()

    return pl.kernel(
        body,
        out_type=[jax.ShapeDtypeStruct((NPAD, 128), jnp.float32)] * n_tab,
        mesh=mesh,
        compiler_params=pltpu.CompilerParams(use_tc_tiling_on_sc=False),
        scratch_types=[
            pltpu.VMEM_SHARED((NPAD, 128), jnp.float32),
            pltpu.VMEM((CH, EB), jnp.int32),
            pltpu.VMEM((CH, EB), jnp.int32),
            pltpu.VMEM((EB, 128), jnp.float32),
            pltpu.VMEM((EB, 128), jnp.float32),
            pltpu.VMEM((ZB, 128), jnp.float32),
            pltpu.SemaphoreType.DMA,
            pltpu.SemaphoreType.DMA,
            pltpu.SemaphoreType.DMA,
            pltpu.SemaphoreType.DMA,
        ],
    )


_sc_agg2 = _make_sc_agg(1)
_sc_agg4 = _make_sc_agg(2)


def kernel(x_phys, x_sem, edge_index, params):
    src3 = edge_index[0].reshape(NS, NITER, EB)
    dst3 = edge_index[1].reshape(NS, NITER, EB)
    zeros = jnp.zeros((ZB, 128), jnp.float32)
    pp = params['phys_enc']
    ps = params['sem_enc']

    agg0_p, agg0_s = _sc_agg2(x_phys, x_sem, src3, dst3, zeros)
    h1l_p, h1r_p = _mlp0(x_phys, agg0_p[:N], pp[0]['W1'], pp[0]['b1'],
                         pp[0]['W2'], pp[0]['b2'])
    h1l_s, h1r_s = _mlp0(x_sem, agg0_s[:N], ps[0]['W1'], ps[0]['b1'],
                         ps[0]['W2'], ps[0]['b2'])
    a1l_p, a1l_s, a1r_p, a1r_s = (
        a[:N] for a in _sc_agg4(h1l_p, h1l_s, h1r_p, h1r_s,
                                src3, dst3, zeros))
    h2_p = _mlp1(h1l_p, h1r_p, a1l_p, a1r_p, pp[1]['W1'], pp[1]['b1'],
                 pp[1]['W2'], pp[1]['b2'])
    h2_s = _mlp1(h1l_s, h1r_s, a1l_s, a1r_s, ps[1]['W1'], ps[1]['b1'],
                 ps[1]['W2'], ps[1]['b2'])
    z_p = _proj_head(params['phys_proj'], h2_p)
    z_s = _proj_head(params['sem_proj'], h2_s)
    return (h2_p, h2_s, z_p, z_s)


# speedup vs baseline: 1.2317x; 1.0758x over previous
import functools

import jax
import jax.numpy as jnp
from jax import lax
from jax.experimental import pallas as pl
from jax.experimental.pallas import tpu as pltpu
from jax.experimental.pallas import tpu_sc as plsc

N = 10000
E = 320000
IN_DIM = 128
HID = 256
REPR = 128
BK = 1000
G = N // BK

NC = 2
NS = 16
EB = 125
NITER = E // (NS * EB)
CH = 20
ZB = 80
NPAD = 10240
ROWS_PER_SUB = NPAD // NS

_DOT = functools.partial(jnp.dot, preferred_element_type=jnp.float32)


def _row_spec(d):
    return pl.BlockSpec((BK, d), lambda i: (i, 0))


def _full_spec(shape):
    nd = len(shape)
    return pl.BlockSpec(shape, lambda i: (0,) * nd)


def _mlp0_pair_body(xp_ref, aggp_ref, xs_ref, aggs_ref,
                    w1p_ref, b1p_ref, w2p_ref, b2p_ref,
                    w1s_ref, b1s_ref, w2s_ref, b2s_ref,
                    hlp_ref, hrp_ref, hls_ref, hrs_ref):
    for (x_ref, agg_ref, w1_ref, b1_ref, w2_ref, b2_ref, hl_ref,
         hr_ref) in (
            (xp_ref, aggp_ref, w1p_ref, b1p_ref, w2p_ref, b2p_ref,
             hlp_ref, hrp_ref),
            (xs_ref, aggs_ref, w1s_ref, b1s_ref, w2s_ref, b2s_ref,
             hls_ref, hrs_ref)):
        m = x_ref[...] + agg_ref[...]
        a = jnp.maximum(_DOT(m, w1_ref[...]) + b1_ref[...], 0.0)
        h = jnp.maximum(_DOT(a, w2_ref[...]) + b2_ref[...], 0.0)
        hl_ref[...] = h[:, :HID // 2]
        hr_ref[...] = h[:, HID // 2:]


def _mlp0_pair(xp, aggp, xs, aggs, pp, ps):
    return pl.pallas_call(
        _mlp0_pair_body,
        grid=(G,),
        in_specs=[_row_spec(IN_DIM), _row_spec(IN_DIM),
                  _row_spec(IN_DIM), _row_spec(IN_DIM)] +
                 [_full_spec((IN_DIM, HID)), _full_spec((1, HID)),
                  _full_spec((HID, HID)), _full_spec((1, HID))] * 2,
        out_specs=[_row_spec(HID // 2)] * 4,
        out_shape=[jax.ShapeDtypeStruct((N, HID // 2), jnp.float32)] * 4,
    )(xp, aggp, xs, aggs,
      pp['W1'], pp['b1'].reshape(1, -1), pp['W2'], pp['b2'].reshape(1, -1),
      ps['W1'], ps['b1'].reshape(1, -1), ps['W2'], ps['b2'].reshape(1, -1))


def _mlp1_proja_pair_body(hlp_ref, hrp_ref, alp_ref, arp_ref,
                          hls_ref, hrs_ref, als_ref, ars_ref,
                          w1p_ref, b1p_ref, w2p_ref, b2p_ref,
                          w1s_ref, b1s_ref, w2s_ref, b2s_ref,
                          pw1p_ref, pb1p_ref, pw1s_ref, pb1s_ref,
                          h2p_ref, h2s_ref, tp_ref, ts_ref,
                          s1p_ref, s2p_ref, s1s_ref, s2s_ref):
    for (hl_ref, hr_ref, al_ref, ar_ref, w1_ref, b1_ref, w2_ref, b2_ref,
         pw1_ref, pb1_ref, h2_ref, t_ref, s1_ref, s2_ref) in (
            (hlp_ref, hrp_ref, alp_ref, arp_ref, w1p_ref, b1p_ref,
             w2p_ref, b2p_ref, pw1p_ref, pb1p_ref, h2p_ref, tp_ref,
             s1p_ref, s2p_ref),
            (hls_ref, hrs_ref, als_ref, ars_ref, w1s_ref, b1s_ref,
             w2s_ref, b2s_ref, pw1s_ref, pb1s_ref, h2s_ref, ts_ref,
             s1s_ref, s2s_ref)):
        m = jnp.concatenate([hl_ref[...] + al_ref[...],
                             hr_ref[...] + ar_ref[...]], axis=1)
        a = jnp.maximum(_DOT(m, w1_ref[...]) + b1_ref[...], 0.0)
        h2 = _DOT(a, w2_ref[...]) + b2_ref[...]
        h2_ref[...] = h2
        t = _DOT(h2, pw1_ref[...]) + pb1_ref[...]
        t_ref[...] = t
        s1_ref[...] = jnp.sum(t, axis=0, keepdims=True)[None]
        s2_ref[...] = jnp.sum(t * t, axis=0, keepdims=True)[None]


def _mlp1_proja_pair(hlp, hrp, alp, arp, hls, hrs, als, ars, pp, ps,
                     qp, qs):
    return pl.pallas_call(
        _mlp1_proja_pair_body,
        grid=(G,),
        in_specs=[_row_spec(HID // 2)] * 8 +
                 [_full_spec((HID, HID)), _full_spec((1, HID)),
                  _full_spec((HID, REPR)), _full_spec((1, REPR))] * 2 +
                 [_full_spec((REPR, REPR)), _full_spec((1, REPR))] * 2,
        out_specs=[_row_spec(REPR)] * 4 +
                  [pl.BlockSpec((1, 1, REPR), lambda i: (i, 0, 0))] * 4,
        out_shape=[jax.ShapeDtypeStruct((N, REPR), jnp.float32)] * 4 +
                  [jax.ShapeDtypeStruct((G, 1, REPR), jnp.float32)] * 4,
    )(hlp, hrp, alp, arp, hls, hrs, als, ars,
      pp['W1'], pp['b1'].reshape(1, -1), pp['W2'], pp['b2'].reshape(1, -1),
      ps['W1'], ps['b1'].reshape(1, -1), ps['W2'], ps['b2'].reshape(1, -1),
      qp['W1'], qp['b1'].reshape(1, -1), qs['W1'], qs['b1'].reshape(1, -1))


def _projb_pair_body(tp_ref, s1p_ref, s2p_ref, ts_ref, s1s_ref, s2s_ref,
                     gp_ref, bp_ref, w2p_ref, b2p_ref,
                     gs_ref, bs_ref, w2s_ref, b2s_ref,
                     zp_ref, zs_ref):
    for (t_ref, s1_ref, s2_ref, gamma_ref, beta_ref, w2_ref, b2_ref,
         z_ref) in (
            (tp_ref, s1p_ref, s2p_ref, gp_ref, bp_ref, w2p_ref, b2p_ref,
             zp_ref),
            (ts_ref, s1s_ref, s2s_ref, gs_ref, bs_ref, w2s_ref, b2s_ref,
             zs_ref)):
        mean = jnp.sum(s1_ref[...], axis=0) / N
        ex2 = jnp.sum(s2_ref[...], axis=0) / N
        var = ex2 - mean * mean
        norm = gamma_ref[...] * (t_ref[...] - mean) * \
            jax.lax.rsqrt(var + 1e-5) + beta_ref[...]
        z_ref[...] = _DOT(jnp.maximum(norm, 0.0), w2_ref[...]) + b2_ref[...]


def _projb_pair(tp, s1p, s2p, ts, s1s, s2s, qp, qs):
    return pl.pallas_call(
        _projb_pair_body,
        grid=(G,),
        in_specs=[_row_spec(REPR), _full_spec((G, 1, REPR)),
                  _full_spec((G, 1, REPR))] * 2 +
                 [_full_spec((1, REPR)), _full_spec((1, REPR)),
                  _full_spec((REPR, REPR)), _full_spec((1, REPR))] * 2,
        out_specs=[_row_spec(REPR)] * 2,
        out_shape=[jax.ShapeDtypeStruct((N, REPR), jnp.float32)] * 2,
    )(tp, s1p, s2p, ts, s1s, s2s,
      qp['gamma'].reshape(1, -1), qp['beta'].reshape(1, -1), qp['W2'],
      qp['b2'].reshape(1, -1),
      qs['gamma'].reshape(1, -1), qs['beta'].reshape(1, -1), qs['W2'],
      qs['b2'].reshape(1, -1))


@functools.lru_cache(maxsize=None)
def _make_sc_agg(nrounds):
    n_tab = NC * nrounds
    mesh = plsc.VectorSubcoreMesh(core_axis_name="c", subcore_axis_name="s")

    def body(*refs):
        tabs = refs[:n_tab]
        src_hbm, dst_hbm, zeros_hbm = refs[n_tab:n_tab + 3]
        outs = refs[n_tab + 3:2 * n_tab + 3]
        (acc, src_v, dst_v, rows_v, rows_b, zbuf, sem, sem_b, sem_sa,
         sem_sb) = refs[2 * n_tab + 3:]
        c = lax.axis_index("c")
        s = lax.axis_index("s")
        row0 = s * ROWS_PER_SUB
        pltpu.sync_copy(zeros_hbm, zbuf)
        nchunk = ROWS_PER_SUB // ZB
        for r in range(nrounds):
            for k in range(nchunk):
                pltpu.sync_copy(zbuf, acc.at[pl.ds(row0 + k * ZB, ZB)])
            plsc.subcore_barrier()
            for ci in range(NC):
                tab = tabs[r * NC + ci]

                @pl.when(c == ci)
                def _():
                    def chunk(g, carry):
                        pltpu.sync_copy(src_hbm.at[s, pl.ds(g * CH, CH)],
                                        src_v)
                        pltpu.sync_copy(dst_hbm.at[s, pl.ds(g * CH, CH)],
                                        dst_v)
                        pltpu.async_copy(tab.at[src_v.at[0]], rows_v, sem)

                        def it(i, carry2):
                            j0 = 2 * i
                            pltpu.make_async_copy(
                                tab.at[src_v.at[j0]], rows_v, sem).wait()

                            @pl.when(i > 0)
                            def _():
                                pltpu.make_async_copy(
                                    rows_b, acc.at[dst_v.at[j0 - 1]],
                                    sem_sb).wait()
                            pltpu.async_copy(rows_v,
                                             acc.at[dst_v.at[j0]],
                                             sem_sa, add=True)
                            pltpu.async_copy(
                                tab.at[src_v.at[j0 + 1]], rows_b, sem_b)
                            pltpu.make_async_copy(
                                tab.at[src_v.at[j0 + 1]], rows_b,
                                sem_b).wait()
                            pltpu.async_copy(rows_b,
                                             acc.at[dst_v.at[j0 + 1]],
                                             sem_sb, add=True)
                            pltpu.make_async_copy(
                                rows_v, acc.at[dst_v.at[j0]],
                                sem_sa).wait()

                            @pl.when(i < CH // 2 - 1)
                            def _():
                                pltpu.async_copy(
                                    tab.at[src_v.at[j0 + 2]], rows_v, sem)
                            return carry2
                        lax.fori_loop(0, CH // 2, it, 0)
                        pltpu.make_async_copy(
                            rows_b, acc.at[dst_v.at[CH - 1]],
                            sem_sb).wait()
                        return carry
                    lax.fori_loop(0, NITER // CH, chunk, 0)
            plsc.subcore_barrier()
            for ci in range(NC):
                out = outs[r * NC + ci]

                @pl.when(c == ci)
                def _():
                    for k in range(nchunk):
                        pltpu.sync_copy(acc.at[pl.ds(row0 + k * ZB, ZB)],
                                        rows_v.at[pl.ds(0, ZB)])
                        pltpu.sync_copy(rows_v.at[pl.ds(0, ZB)],
                                        out.at[pl.ds(row0 + k * ZB, ZB)])
            if r + 1 < nrounds:
                plsc.subcore_barrier()

    return pl.kernel(
        body,
        out_type=[jax.ShapeDtypeStruct((NPAD, 128), jnp.float32)] * n_tab,
        mesh=mesh,
        compiler_params=pltpu.CompilerParams(use_tc_tiling_on_sc=False),
        scratch_types=[
            pltpu.VMEM_SHARED((NPAD, 128), jnp.float32),
            pltpu.VMEM((CH, EB), jnp.int32),
            pltpu.VMEM((CH, EB), jnp.int32),
            pltpu.VMEM((EB, 128), jnp.float32),
            pltpu.VMEM((EB, 128), jnp.float32),
            pltpu.VMEM((ZB, 128), jnp.float32),
            pltpu.SemaphoreType.DMA,
            pltpu.SemaphoreType.DMA,
            pltpu.SemaphoreType.DMA,
            pltpu.SemaphoreType.DMA,
        ],
    )


def _sc_agg2(*args):
    return _make_sc_agg(1)(*args)


def _sc_agg4(*args):
    return _make_sc_agg(2)(*args)


def kernel(x_phys, x_sem, edge_index, params):
    src3 = edge_index[0].reshape(NS, NITER, EB)
    dst3 = edge_index[1].reshape(NS, NITER, EB)
    zeros = jnp.zeros((ZB, 128), jnp.float32)
    pp = params['phys_enc']
    ps = params['sem_enc']

    agg0_p, agg0_s = _sc_agg2(x_phys, x_sem, src3, dst3, zeros)
    h1l_p, h1r_p, h1l_s, h1r_s = _mlp0_pair(x_phys, agg0_p, x_sem,
                                            agg0_s, pp[0], ps[0])
    a1l_p, a1l_s, a1r_p, a1r_s = _sc_agg4(h1l_p, h1l_s, h1r_p, h1r_s,
                                          src3, dst3, zeros)
    qp = params['phys_proj']
    qs = params['sem_proj']
    (h2_p, h2_s, t_p, t_s, s1p, s2p, s1s, s2s) = _mlp1_proja_pair(
        h1l_p, h1r_p, a1l_p, a1r_p, h1l_s, h1r_s, a1l_s, a1r_s,
        pp[1], ps[1], qp, qs)
    z_p, z_s = _projb_pair(t_p, s1p, s2p, t_s, s1s, s2s, qp, qs)
    return (h2_p, h2_s, z_p, z_s)


# speedup vs baseline: 1.2572x; 1.0207x over previous
import functools

import jax
import jax.numpy as jnp
from jax import lax
from jax.experimental import pallas as pl
from jax.experimental.pallas import tpu as pltpu
from jax.experimental.pallas import tpu_sc as plsc

N = 10000
E = 320000
IN_DIM = 128
HID = 256
REPR = 128
BK = 1000
G = N // BK

NC = 2
NS = 16
EB = 125
NITER = E // (NS * EB)
CH = 40
ZB = 80
NPAD = 10240
ROWS_PER_SUB = NPAD // NS

_DOT = functools.partial(jnp.dot, preferred_element_type=jnp.float32)


def _row_spec(d):
    return pl.BlockSpec((BK, d), lambda i: (i, 0))


def _full_spec(shape):
    nd = len(shape)
    return pl.BlockSpec(shape, lambda i: (0,) * nd)


def _mlp0_pair_body(xp_ref, aggp_ref, xs_ref, aggs_ref,
                    w1p_ref, b1p_ref, w2p_ref, b2p_ref,
                    w1s_ref, b1s_ref, w2s_ref, b2s_ref,
                    hlp_ref, hrp_ref, hls_ref, hrs_ref):
    for (x_ref, agg_ref, w1_ref, b1_ref, w2_ref, b2_ref, hl_ref,
         hr_ref) in (
            (xp_ref, aggp_ref, w1p_ref, b1p_ref, w2p_ref, b2p_ref,
             hlp_ref, hrp_ref),
            (xs_ref, aggs_ref, w1s_ref, b1s_ref, w2s_ref, b2s_ref,
             hls_ref, hrs_ref)):
        m = x_ref[...] + agg_ref[...]
        a = jnp.maximum(_DOT(m, w1_ref[...]) + b1_ref[...], 0.0)
        h = jnp.maximum(_DOT(a, w2_ref[...]) + b2_ref[...], 0.0)
        hl_ref[...] = h[:, :HID // 2]
        hr_ref[...] = h[:, HID // 2:]


def _mlp0_pair(xp, aggp, xs, aggs, pp, ps):
    return pl.pallas_call(
        _mlp0_pair_body,
        grid=(G,),
        in_specs=[_row_spec(IN_DIM), _row_spec(IN_DIM),
                  _row_spec(IN_DIM), _row_spec(IN_DIM)] +
                 [_full_spec((IN_DIM, HID)), _full_spec((1, HID)),
                  _full_spec((HID, HID)), _full_spec((1, HID))] * 2,
        out_specs=[_row_spec(HID // 2)] * 4,
        out_shape=[jax.ShapeDtypeStruct((N, HID // 2), jnp.float32)] * 4,
    )(xp, aggp, xs, aggs,
      pp['W1'], pp['b1'].reshape(1, -1), pp['W2'], pp['b2'].reshape(1, -1),
      ps['W1'], ps['b1'].reshape(1, -1), ps['W2'], ps['b2'].reshape(1, -1))


def _mlp1_proja_pair_body(hlp_ref, hrp_ref, alp_ref, arp_ref,
                          hls_ref, hrs_ref, als_ref, ars_ref,
                          w1p_ref, b1p_ref, w2p_ref, b2p_ref,
                          w1s_ref, b1s_ref, w2s_ref, b2s_ref,
                          pw1p_ref, pb1p_ref, pw1s_ref, pb1s_ref,
                          h2p_ref, h2s_ref, tp_ref, ts_ref,
                          s1p_ref, s2p_ref, s1s_ref, s2s_ref):
    for (hl_ref, hr_ref, al_ref, ar_ref, w1_ref, b1_ref, w2_ref, b2_ref,
         pw1_ref, pb1_ref, h2_ref, t_ref, s1_ref, s2_ref) in (
            (hlp_ref, hrp_ref, alp_ref, arp_ref, w1p_ref, b1p_ref,
             w2p_ref, b2p_ref, pw1p_ref, pb1p_ref, h2p_ref, tp_ref,
             s1p_ref, s2p_ref),
            (hls_ref, hrs_ref, als_ref, ars_ref, w1s_ref, b1s_ref,
             w2s_ref, b2s_ref, pw1s_ref, pb1s_ref, h2s_ref, ts_ref,
             s1s_ref, s2s_ref)):
        m = jnp.concatenate([hl_ref[...] + al_ref[...],
                             hr_ref[...] + ar_ref[...]], axis=1)
        a = jnp.maximum(_DOT(m, w1_ref[...]) + b1_ref[...], 0.0)
        h2 = _DOT(a, w2_ref[...]) + b2_ref[...]
        h2_ref[...] = h2
        t = _DOT(h2, pw1_ref[...]) + pb1_ref[...]
        t_ref[...] = t
        s1_ref[...] = jnp.sum(t, axis=0, keepdims=True)[None]
        s2_ref[...] = jnp.sum(t * t, axis=0, keepdims=True)[None]


def _mlp1_proja_pair(hlp, hrp, alp, arp, hls, hrs, als, ars, pp, ps,
                     qp, qs):
    return pl.pallas_call(
        _mlp1_proja_pair_body,
        grid=(G,),
        in_specs=[_row_spec(HID // 2)] * 8 +
                 [_full_spec((HID, HID)), _full_spec((1, HID)),
                  _full_spec((HID, REPR)), _full_spec((1, REPR))] * 2 +
                 [_full_spec((REPR, REPR)), _full_spec((1, REPR))] * 2,
        out_specs=[_row_spec(REPR)] * 4 +
                  [pl.BlockSpec((1, 1, REPR), lambda i: (i, 0, 0))] * 4,
        out_shape=[jax.ShapeDtypeStruct((N, REPR), jnp.float32)] * 4 +
                  [jax.ShapeDtypeStruct((G, 1, REPR), jnp.float32)] * 4,
    )(hlp, hrp, alp, arp, hls, hrs, als, ars,
      pp['W1'], pp['b1'].reshape(1, -1), pp['W2'], pp['b2'].reshape(1, -1),
      ps['W1'], ps['b1'].reshape(1, -1), ps['W2'], ps['b2'].reshape(1, -1),
      qp['W1'], qp['b1'].reshape(1, -1), qs['W1'], qs['b1'].reshape(1, -1))


def _projb_pair_body(tp_ref, s1p_ref, s2p_ref, ts_ref, s1s_ref, s2s_ref,
                     gp_ref, bp_ref, w2p_ref, b2p_ref,
                     gs_ref, bs_ref, w2s_ref, b2s_ref,
                     zp_ref, zs_ref):
    for (t_ref, s1_ref, s2_ref, gamma_ref, beta_ref, w2_ref, b2_ref,
         z_ref) in (
            (tp_ref, s1p_ref, s2p_ref, gp_ref, bp_ref, w2p_ref, b2p_ref,
             zp_ref),
            (ts_ref, s1s_ref, s2s_ref, gs_ref, bs_ref, w2s_ref, b2s_ref,
             zs_ref)):
        mean = jnp.sum(s1_ref[...], axis=0) / N
        ex2 = jnp.sum(s2_ref[...], axis=0) / N
        var = ex2 - mean * mean
        norm = gamma_ref[...] * (t_ref[...] - mean) * \
            jax.lax.rsqrt(var + 1e-5) + beta_ref[...]
        z_ref[...] = _DOT(jnp.maximum(norm, 0.0), w2_ref[...]) + b2_ref[...]


def _projb_pair(tp, s1p, s2p, ts, s1s, s2s, qp, qs):
    return pl.pallas_call(
        _projb_pair_body,
        grid=(G,),
        in_specs=[_row_spec(REPR), _full_spec((G, 1, REPR)),
                  _full_spec((G, 1, REPR))] * 2 +
                 [_full_spec((1, REPR)), _full_spec((1, REPR)),
                  _full_spec((REPR, REPR)), _full_spec((1, REPR))] * 2,
        out_specs=[_row_spec(REPR)] * 2,
        out_shape=[jax.ShapeDtypeStruct((N, REPR), jnp.float32)] * 2,
    )(tp, s1p, s2p, ts, s1s, s2s,
      qp['gamma'].reshape(1, -1), qp['beta'].reshape(1, -1), qp['W2'],
      qp['b2'].reshape(1, -1),
      qs['gamma'].reshape(1, -1), qs['beta'].reshape(1, -1), qs['W2'],
      qs['b2'].reshape(1, -1))


@functools.lru_cache(maxsize=None)
def _make_sc_agg(nrounds):
    n_tab = NC * nrounds
    mesh = plsc.VectorSubcoreMesh(core_axis_name="c", subcore_axis_name="s")

    def body(*refs):
        tabs = refs[:n_tab]
        src_hbm, dst_hbm, zeros_hbm = refs[n_tab:n_tab + 3]
        outs = refs[n_tab + 3:2 * n_tab + 3]
        (acc, src_v, dst_v, rows_v, rows_b, sem, sem_b, sem_sa,
         sem_sb) = refs[2 * n_tab + 3:]
        c = lax.axis_index("c")
        s = lax.axis_index("s")
        row0 = s * ROWS_PER_SUB
        nchunk = ROWS_PER_SUB // ZB
        for r in range(nrounds):
            pltpu.sync_copy(zeros_hbm, acc.at[pl.ds(row0, ROWS_PER_SUB)])
            plsc.subcore_barrier()
            for ci in range(NC):
                tab = tabs[r * NC + ci]

                @pl.when(c == ci)
                def _():
                    def chunk(g, carry):
                        pltpu.sync_copy(src_hbm.at[s, pl.ds(g * CH, CH)],
                                        src_v)
                        pltpu.sync_copy(dst_hbm.at[s, pl.ds(g * CH, CH)],
                                        dst_v)
                        pltpu.async_copy(tab.at[src_v.at[0]], rows_v, sem)

                        def it(i, carry2):
                            j0 = 2 * i
                            pltpu.make_async_copy(
                                tab.at[src_v.at[j0]], rows_v, sem).wait()

                            @pl.when(i > 0)
                            def _():
                                pltpu.make_async_copy(
                                    rows_b, acc.at[dst_v.at[j0 - 1]],
                                    sem_sb).wait()
                            pltpu.async_copy(rows_v,
                                             acc.at[dst_v.at[j0]],
                                             sem_sa, add=True)
                            pltpu.async_copy(
                                tab.at[src_v.at[j0 + 1]], rows_b, sem_b)
                            pltpu.make_async_copy(
                                tab.at[src_v.at[j0 + 1]], rows_b,
                                sem_b).wait()
                            pltpu.async_copy(rows_b,
                                             acc.at[dst_v.at[j0 + 1]],
                                             sem_sb, add=True)
                            pltpu.make_async_copy(
                                rows_v, acc.at[dst_v.at[j0]],
                                sem_sa).wait()

                            @pl.when(i < CH // 2 - 1)
                            def _():
                                pltpu.async_copy(
                                    tab.at[src_v.at[j0 + 2]], rows_v, sem)
                            return carry2
                        lax.fori_loop(0, CH // 2, it, 0)
                        pltpu.make_async_copy(
                            rows_b, acc.at[dst_v.at[CH - 1]],
                            sem_sb).wait()
                        return carry
                    lax.fori_loop(0, NITER // CH, chunk, 0)
            plsc.subcore_barrier()
            for ci in range(NC):
                out = outs[r * NC + ci]

                @pl.when(c == ci)
                def _():
                    pltpu.sync_copy(acc.at[pl.ds(row0, ROWS_PER_SUB)],
                                    out.at[pl.ds(row0, ROWS_PER_SUB)])
            if r + 1 < nrounds:
                plsc.subcore_barrier()

    return pl.kernel(
        body,
        out_type=[jax.ShapeDtypeStruct((NPAD, 128), jnp.float32)] * n_tab,
        mesh=mesh,
        compiler_params=pltpu.CompilerParams(use_tc_tiling_on_sc=False),
        scratch_types=[
            pltpu.VMEM_SHARED((NPAD, 128), jnp.float32),
            pltpu.VMEM((CH, EB), jnp.int32),
            pltpu.VMEM((CH, EB), jnp.int32),
            pltpu.VMEM((EB, 128), jnp.float32),
            pltpu.VMEM((EB, 128), jnp.float32),
            pltpu.SemaphoreType.DMA,
            pltpu.SemaphoreType.DMA,
            pltpu.SemaphoreType.DMA,
            pltpu.SemaphoreType.DMA,
        ],
    )


def _sc_agg2(*args):
    return _make_sc_agg(1)(*args)


def _sc_agg4(*args):
    return _make_sc_agg(2)(*args)


def kernel(x_phys, x_sem, edge_index, params):
    src3 = edge_index[0].reshape(NS, NITER, EB)
    dst3 = edge_index[1].reshape(NS, NITER, EB)
    zeros = jnp.zeros((ROWS_PER_SUB, 128), jnp.float32)
    pp = params['phys_enc']
    ps = params['sem_enc']

    agg0_p, agg0_s = _sc_agg2(x_phys, x_sem, src3, dst3, zeros)
    h1l_p, h1r_p, h1l_s, h1r_s = _mlp0_pair(x_phys, agg0_p, x_sem,
                                            agg0_s, pp[0], ps[0])
    a1l_p, a1l_s, a1r_p, a1r_s = _sc_agg4(h1l_p, h1l_s, h1r_p, h1r_s,
                                          src3, dst3, zeros)
    qp = params['phys_proj']
    qs = params['sem_proj']
    (h2_p, h2_s, t_p, t_s, s1p, s2p, s1s, s2s) = _mlp1_proja_pair(
        h1l_p, h1r_p, a1l_p, a1r_p, h1l_s, h1r_s, a1l_s, a1r_s,
        pp[1], ps[1], qp, qs)
    z_p, z_s = _projb_pair(t_p, s1p, s2p, t_s, s1s, s2s, qp, qs)
    return (h2_p, h2_s, z_p, z_s)
